# baseline (speedup 1.0000x reference)
import functools

import jax
import jax.numpy as jnp
from jax import lax
from jax.experimental import pallas as pl
from jax.experimental.pallas import tpu as pltpu
from jax.experimental.pallas import tpu_sc as plsc

N_NODES = 10000
N_EDGES = 320000
D = 128

NC = 2
NS = 16
NW = NC * NS
C = 96
CH = 108
EPW = C * CH
E_PAD = NW * EPW - N_EDGES
NPAD = 10240
ROWS_PER_TILE = NPAD // NS

_SQRT_HALF = 0.7071067811865476


def _gelu_exact(v):
    return 0.5 * v * (1.0 + lax.erf(v * _SQRT_HALF))


DEG_ROWS = 2 * NPAD // 128


def _sc_body(x_hbm, idx_hbm, z128_hbm,
             out_nbx, out_deg,
             idx0, idx1, rows0, rows1, deg_v, accx,
             gsem0, gsem1, ssem0, ssem1):
    cid = lax.axis_index("c")
    sid = lax.axis_index("s")
    wid = sid * NC + cid

    base = sid * ROWS_PER_TILE
    pltpu.sync_copy(z128_hbm, accx.at[pl.ds(base, ROWS_PER_TILE)])
    pltpu.sync_copy(z128_hbm.at[pl.ds(0, DEG_ROWS)], deg_v)
    plsc.subcore_barrier()

    iota16 = lax.iota(jnp.int32, 16)
    colw = jnp.bitwise_and(iota16, 1)
    pair = jnp.right_shift(iota16, 1)
    pair_masks = [pair == kk for kk in range(8)]
    ones16 = jnp.full((16,), 1.0, dtype=jnp.float32)

    def hist(idx_v):
        def h(kk, carry2):
            sv = idx_v[0, pl.ds(kk * 16, 16)]
            flat = jnp.left_shift(sv, 1) + colw
            r = jnp.right_shift(flat, 7)
            cc = jnp.bitwise_and(flat, 127)
            for mm in pair_masks:
                plsc.addupdate_scatter(deg_v, [r, cc], ones16, mask=mm)
            return carry2

        lax.fori_loop(0, C // 16, h, 0)

    cbase = wid * CH

    pltpu.sync_copy(idx_hbm.at[cbase], idx0)
    pltpu.async_copy(x_hbm.at[idx0.at[1]], rows0, gsem0)

    def pair_body(p, carry):
        j = cbase + 2 * p
        pltpu.make_async_copy(x_hbm.at[idx0.at[1]], rows0, gsem0).wait()

        pltpu.sync_copy(idx_hbm.at[j + 1], idx1)
        pltpu.async_copy(x_hbm.at[idx1.at[1]], rows1, gsem1)
        hist(idx0)

        pltpu.make_async_copy(x_hbm.at[idx1.at[1]], rows1, gsem1).wait()

        @pl.when(p < CH // 2 - 1)
        def _():
            pltpu.sync_copy(idx_hbm.at[j + 2], idx0)
            pltpu.async_copy(x_hbm.at[idx0.at[1]], rows0, gsem0)

        hist(idx1)
        return carry

    lax.fori_loop(0, CH // 2, pair_body, 0)
    plsc.subcore_barrier()

    pltpu.sync_copy(accx.at[pl.ds(base, ROWS_PER_TILE)],
                    out_nbx.at[cid, pl.ds(base, ROWS_PER_TILE)])
    pltpu.sync_copy(deg_v, out_deg.at[cid, sid])


@functools.lru_cache(maxsize=1)
def _build_sc_scatter():
    return pl.kernel(
        _sc_body,
        out_type=(jax.ShapeDtypeStruct((NC, NPAD, D), jnp.float32),
                  jax.ShapeDtypeStruct((NC, NS, DEG_ROWS, 128), jnp.float32)),
        mesh=plsc.VectorSubcoreMesh(core_axis_name="c", subcore_axis_name="s",
                                    num_cores=NC, num_subcores=NS),
        scratch_types=(
            pltpu.VMEM((2, C), jnp.int32),
            pltpu.VMEM((2, C), jnp.int32),
            pltpu.VMEM((C, D), jnp.float32),
            pltpu.VMEM((C, D), jnp.float32),
            pltpu.VMEM((DEG_ROWS, 128), jnp.float32),
            pltpu.VMEM_SHARED((NPAD, D), jnp.float32),
            pltpu.SemaphoreType.DMA,
            pltpu.SemaphoreType.DMA,
            pltpu.SemaphoreType.DMA,
            pltpu.SemaphoreType.DMA,
        ),
        compiler_params=pltpu.CompilerParams(needs_layout_passes=False),
    )


BLK = 400


def _dense_body(x_ref, nbp_ref, degp_ref, wn_ref, bn_ref,
                we1_ref, we2_ref, be_ref, wu1_ref, wu2_ref, wu3_ref, bu_ref,
                out_ref):
    f32 = jnp.float32
    xb = x_ref[...]
    wn = wn_ref[...]
    node = jnp.dot(xb, wn, preferred_element_type=f32) + bn_ref[...]
    nbx = nbp_ref[0] + nbp_ref[1]
    deg = jnp.sum(degp_ref[...], axis=1, keepdims=True)
    nb = jnp.dot(nbx, wn, preferred_element_type=f32) + deg * bn_ref[...]
    es = (deg * (jnp.dot(node, we1_ref[...], preferred_element_type=f32)
                 + be_ref[...])
          + jnp.dot(nb, we2_ref[...], preferred_element_type=f32))
    out_ref[...] = (jnp.dot(_gelu_exact(node), wu1_ref[...],
                            preferred_element_type=f32)
                    + jnp.dot(_gelu_exact(nb), wu2_ref[...],
                              preferred_element_type=f32)
                    + jnp.dot(_gelu_exact(es), wu3_ref[...],
                              preferred_element_type=f32)
                    + bu_ref[...])


def _w_spec():
    return pl.BlockSpec((D, D), lambda i: (0, 0))


def _b_spec():
    return pl.BlockSpec((1, D), lambda i: (0, 0))


_dense_call = pl.pallas_call(
    _dense_body,
    grid=(N_NODES // BLK,),
    in_specs=[
        pl.BlockSpec((BLK, D), lambda i: (i, 0)),
        pl.BlockSpec((NC, BLK, D), lambda i: (0, i, 0)),
        pl.BlockSpec((BLK, 2 * NC * NS), lambda i: (i, 0)),
        _w_spec(), _b_spec(),
        _w_spec(), _w_spec(), _b_spec(),
        _w_spec(), _w_spec(), _w_spec(), _b_spec(),
    ],
    out_specs=pl.BlockSpec((BLK, D), lambda i: (i, 0)),
    out_shape=jax.ShapeDtypeStruct((N_NODES, D), jnp.float32),
)


def kernel(input_embeddings, edge_index, W_node, b_node, W_edge, b_edge,
           W_upd, b_upd):
    x = input_embeddings
    src = jnp.concatenate([edge_index[0].astype(jnp.int32),
                           jnp.full((E_PAD,), NPAD - 1, jnp.int32)])
    dst = jnp.concatenate([edge_index[1].astype(jnp.int32),
                           jnp.zeros((E_PAD,), jnp.int32)])
    idx = jnp.stack([src.reshape(NW * CH, C),
                     dst.reshape(NW * CH, C)], axis=1)
    z128 = jnp.zeros((ROWS_PER_TILE, D), jnp.float32)

    nbx_parts, deg_raw = _build_sc_scatter()(x, idx, z128)
    deg_parts = (deg_raw.reshape(NC * NS, NPAD, 2)
                 .transpose(1, 0, 2).reshape(NPAD, 2 * NC * NS))

    out = _dense_call(
        x, nbx_parts, deg_parts,
        W_node.T, b_node.reshape(1, D),
        W_edge[:, :D].T, W_edge[:, D:].T, b_edge.reshape(1, D),
        W_upd[:, :D].T, W_upd[:, D:2 * D].T, W_upd[:, 2 * D:].T,
        b_upd.reshape(1, D),
    )
    return out


# speedup vs baseline: 1.4391x; 1.4391x over previous
import functools

import jax
import jax.numpy as jnp
from jax import lax
from jax.experimental import pallas as pl
from jax.experimental.pallas import tpu as pltpu
from jax.experimental.pallas import tpu_sc as plsc

N_NODES = 10000
N_EDGES = 320000
D = 128

NC = 2
NS = 16
NW = NC * NS
C = 128
CH = 80
CHG = CH // 8
EPW = C * CH
E_PAD = NW * EPW - N_EDGES
NPAD = 10240
ROWS_PER_TILE = NPAD // NS

_SQRT_HALF = 0.7071067811865476


def _gelu_exact(v):
    return 0.5 * v * (1.0 + lax.erf(v * _SQRT_HALF))


DEG_ROWS = 2 * NPAD // 128


def _sc_group(x_hbm, accx, idx, rows0, rows1, gsem0, gsem1, ssem0, ssem1):
    rows = (rows0, rows1)
    gsem = (gsem0, gsem1)
    ssem = (ssem0, ssem1)
    pltpu.make_async_copy(rows0, accx.at[idx.at[0]], ssem0).wait()
    pltpu.async_copy(x_hbm.at[idx.at[1]], rows0, gsem0)
    for c in range(1, 8):
        b = c % 2
        pltpu.make_async_copy(rows[b], accx.at[idx.at[0]], ssem[b]).wait()
        pltpu.async_copy(x_hbm.at[idx.at[2 * c + 1]], rows[b], gsem[b])
        pltpu.make_async_copy(x_hbm.at[idx.at[0]], rows[1 - b],
                              gsem[1 - b]).wait()
        pltpu.async_copy(rows[1 - b], accx.at[idx.at[2 * (c - 1)]],
                         ssem[1 - b], add=True)
    pltpu.make_async_copy(x_hbm.at[idx.at[0]], rows1, gsem1).wait()
    pltpu.async_copy(rows1, accx.at[idx.at[14]], ssem1, add=True)


def _sc_body(x_hbm, idx_hbm, z128_hbm,
             out_nbx,
             idxa, idxb, rows0, rows1, accx,
             gsem0, gsem1, ssem0, ssem1, isema, isemb):
    cid = lax.axis_index("c")
    sid = lax.axis_index("s")
    wid = sid * NC + cid

    base = sid * ROWS_PER_TILE
    pltpu.sync_copy(z128_hbm, accx.at[pl.ds(base, ROWS_PER_TILE)])
    plsc.subcore_barrier()

    gbase = wid * CHG

    pltpu.sync_copy(idx_hbm.at[gbase], idxa)
    pltpu.sync_copy(z128_hbm.at[pl.ds(0, C)], rows0)
    pltpu.sync_copy(z128_hbm.at[pl.ds(0, C)], rows1)
    pltpu.async_copy(rows0, accx.at[idxa.at[0]], ssem0, add=True)
    pltpu.async_copy(rows1, accx.at[idxa.at[0]], ssem1, add=True)

    def pair_body(p, carry):
        @pl.when(p > 0)
        def _():
            pltpu.make_async_copy(idx_hbm.at[gbase], idxa, isema).wait()

        pltpu.async_copy(idx_hbm.at[gbase + 2 * p + 1], idxb, isemb)
        _sc_group(x_hbm, accx, idxa, rows0, rows1,
                  gsem0, gsem1, ssem0, ssem1)
        pltpu.make_async_copy(idx_hbm.at[gbase], idxb, isemb).wait()
        _sc_group(x_hbm, accx, idxb, rows0, rows1,
                  gsem0, gsem1, ssem0, ssem1)

        @pl.when(p < CHG // 2 - 1)
        def _():
            pltpu.async_copy(idx_hbm.at[gbase + 2 * p + 2], idxa, isema)

        return carry

    lax.fori_loop(0, CHG // 2, pair_body, 0)
    pltpu.make_async_copy(rows0, accx.at[idxb.at[0]], ssem0).wait()
    pltpu.make_async_copy(rows1, accx.at[idxb.at[0]], ssem1).wait()
    plsc.subcore_barrier()

    pltpu.sync_copy(accx.at[pl.ds(base, ROWS_PER_TILE)],
                    out_nbx.at[cid, pl.ds(base, ROWS_PER_TILE)])


def _sc_deg_body(idx_hbm, z128_hbm, out_deg, idxa, deg_v):
    cid = lax.axis_index("c")
    sid = lax.axis_index("s")
    wid = sid * NC + cid

    pltpu.sync_copy(z128_hbm.at[pl.ds(0, DEG_ROWS)], deg_v)

    iota16 = lax.iota(jnp.int32, 16)
    colw = jnp.bitwise_and(iota16, 1)
    pairid = jnp.right_shift(iota16, 1)
    pair_masks = [pairid == kk for kk in range(8)]
    ones16 = jnp.full((16,), 1.0, dtype=jnp.float32)

    gbase = wid * CHG

    def gbody(g, carry):
        pltpu.sync_copy(idx_hbm.at[gbase + g], idxa)
        for c in range(8):
            def h(kk, carry2):
                sv = idxa[2 * c, pl.ds(kk * 16, 16)]
                flat = jnp.left_shift(sv, 1) + colw
                r = jnp.right_shift(flat, 7)
                cc = jnp.bitwise_and(flat, 127)
                for mm in pair_masks:
                    plsc.addupdate_scatter(deg_v, [r, cc], ones16, mask=mm)
                return carry2

            lax.fori_loop(0, C // 16, h, 0)
        return carry

    lax.fori_loop(0, CHG, gbody, 0)
    pltpu.sync_copy(deg_v, out_deg.at[cid, sid])


@functools.lru_cache(maxsize=1)
def _build_sc_scatter():
    return pl.kernel(
        _sc_body,
        out_type=jax.ShapeDtypeStruct((NC, NPAD, D), jnp.float32),
        mesh=plsc.VectorSubcoreMesh(core_axis_name="c", subcore_axis_name="s",
                                    num_cores=NC, num_subcores=NS),
        scratch_types=(
            pltpu.VMEM((16, C), jnp.int32),
            pltpu.VMEM((16, C), jnp.int32),
            pltpu.VMEM((C, D), jnp.float32),
            pltpu.VMEM((C, D), jnp.float32),
            pltpu.VMEM_SHARED((NPAD, D), jnp.float32),
            pltpu.SemaphoreType.DMA,
            pltpu.SemaphoreType.DMA,
            pltpu.SemaphoreType.DMA,
            pltpu.SemaphoreType.DMA,
            pltpu.SemaphoreType.DMA,
            pltpu.SemaphoreType.DMA,
        ),
        compiler_params=pltpu.CompilerParams(needs_layout_passes=False),
    )


@functools.lru_cache(maxsize=1)
def _build_sc_deg():
    return pl.kernel(
        _sc_deg_body,
        out_type=jax.ShapeDtypeStruct((NC, NS, DEG_ROWS, 128), jnp.float32),
        mesh=plsc.VectorSubcoreMesh(core_axis_name="c", subcore_axis_name="s",
                                    num_cores=NC, num_subcores=NS),
        scratch_types=(
            pltpu.VMEM((16, C), jnp.int32),
            pltpu.VMEM((DEG_ROWS, 128), jnp.float32),
        ),
        compiler_params=pltpu.CompilerParams(needs_layout_passes=False),
    )


BLK = 400


def _dense_body(x_ref, nbp_ref, degp_ref, wn_ref, bn_ref,
                we1_ref, we2_ref, be_ref, wu1_ref, wu2_ref, wu3_ref, bu_ref,
                out_ref):
    f32 = jnp.float32
    xb = x_ref[...]
    wn = wn_ref[...]
    node = jnp.dot(xb, wn, preferred_element_type=f32) + bn_ref[...]
    nbx = nbp_ref[0] + nbp_ref[1]
    deg = jnp.sum(degp_ref[...], axis=1, keepdims=True)
    nb = jnp.dot(nbx, wn, preferred_element_type=f32) + deg * bn_ref[...]
    es = (deg * (jnp.dot(node, we1_ref[...], preferred_element_type=f32)
                 + be_ref[...])
          + jnp.dot(nb, we2_ref[...], preferred_element_type=f32))
    out_ref[...] = (jnp.dot(_gelu_exact(node), wu1_ref[...],
                            preferred_element_type=f32)
                    + jnp.dot(_gelu_exact(nb), wu2_ref[...],
                              preferred_element_type=f32)
                    + jnp.dot(_gelu_exact(es), wu3_ref[...],
                              preferred_element_type=f32)
                    + bu_ref[...])


def _w_spec():
    return pl.BlockSpec((D, D), lambda i: (0, 0))


def _b_spec():
    return pl.BlockSpec((1, D), lambda i: (0, 0))


_dense_call = pl.pallas_call(
    _dense_body,
    grid=(N_NODES // BLK,),
    in_specs=[
        pl.BlockSpec((BLK, D), lambda i: (i, 0)),
        pl.BlockSpec((NC, BLK, D), lambda i: (0, i, 0)),
        pl.BlockSpec((BLK, 2 * NC * NS), lambda i: (i, 0)),
        _w_spec(), _b_spec(),
        _w_spec(), _w_spec(), _b_spec(),
        _w_spec(), _w_spec(), _w_spec(), _b_spec(),
    ],
    out_specs=pl.BlockSpec((BLK, D), lambda i: (i, 0)),
    out_shape=jax.ShapeDtypeStruct((N_NODES, D), jnp.float32),
)


def kernel(input_embeddings, edge_index, W_node, b_node, W_edge, b_edge,
           W_upd, b_upd):
    x = input_embeddings
    src = jnp.concatenate([edge_index[0].astype(jnp.int32),
                           jnp.full((E_PAD,), NPAD - 1, jnp.int32)])
    dst = jnp.concatenate([edge_index[1].astype(jnp.int32),
                           jnp.zeros((E_PAD,), jnp.int32)])
    idx = jnp.stack([src.reshape(NW, CHG, 8, C),
                     dst.reshape(NW, CHG, 8, C)],
                    axis=3).reshape(NW * CHG, 16, C)
    z128 = jnp.zeros((ROWS_PER_TILE, D), jnp.float32)

    nbx_parts = _build_sc_scatter()(x, idx, z128)
    deg_raw = _build_sc_deg()(idx, z128)
    deg_parts = (deg_raw.reshape(NC * NS, NPAD, 2)
                 .transpose(1, 0, 2).reshape(NPAD, 2 * NC * NS))

    out = _dense_call(
        x, nbx_parts, deg_parts,
        W_node.T, b_node.reshape(1, D),
        W_edge[:, :D].T, W_edge[:, D:].T, b_edge.reshape(1, D),
        W_upd[:, :D].T, W_upd[:, D:2 * D].T, W_upd[:, 2 * D:].T,
        b_upd.reshape(1, D),
    )
    return out
